# Initial kernel scaffold; baseline (speedup 1.0000x reference)
import jax
import jax.numpy as jnp
from jax.experimental import pallas as pl


def kernel(embedded, symbol):
    raise NotImplementedError("write your pallas kernel here")


# speedup vs baseline: 1.2585x; 1.2585x over previous
import math

import jax
import jax.numpy as jnp
from jax.experimental import pallas as pl

D_MODEL = 1024
MAX_LENGTH = 8192
PAD_IDX = 0

_TL = 512


def _pe_table():
    position = jnp.arange(0, MAX_LENGTH, dtype=jnp.float32)[:, None]
    scale_factor = -math.log(10000.0) / D_MODEL
    div_term = jnp.exp(jnp.arange(0, D_MODEL, 2, dtype=jnp.float32) * scale_factor)
    pe = jnp.zeros((MAX_LENGTH, D_MODEL), dtype=jnp.float32)
    pe = pe.at[:, 0::2].set(jnp.sin(position * div_term))
    pe = pe.at[:, 1::2].set(jnp.cos(position * div_term))
    return pe


def _body(sym_ref, emb_ref, pe_ref, out_ref):
    mask = (sym_ref[0, 0, :] != PAD_IDX).astype(jnp.float32)
    out_ref[0] = emb_ref[0] + pe_ref[...] * mask[:, None]


def kernel(embedded, symbol):
    B, L = symbol.shape
    pe = _pe_table()
    sym3 = symbol.reshape(B, 1, L)
    n_l = L // _TL
    return pl.pallas_call(
        _body,
        grid=(n_l, B),
        in_specs=[
            pl.BlockSpec((1, 1, _TL), lambda i, b: (b, 0, i)),
            pl.BlockSpec((1, _TL, D_MODEL), lambda i, b: (b, i, 0)),
            pl.BlockSpec((_TL, D_MODEL), lambda i, b: (i, 0)),
        ],
        out_specs=pl.BlockSpec((1, _TL, D_MODEL), lambda i, b: (b, i, 0)),
        out_shape=jax.ShapeDtypeStruct((B, L, D_MODEL), jnp.float32),
    )(sym3, embedded, pe)


# speedup vs baseline: 3.9288x; 3.1219x over previous
import math

import jax
import jax.numpy as jnp
from jax import lax
from jax.experimental import pallas as pl
from jax.experimental.pallas import tpu as pltpu

D_MODEL = 1024
PAD_IDX = 0
_SF = -math.log(10000.0) / D_MODEL

_TL = 512


def _body(sym_ref, emb_ref, out_ref, pe_s):
    i = pl.program_id(0)
    b = pl.program_id(1)

    @pl.when(b == 0)
    def _compute_pe():
        pos = (i * _TL + lax.broadcasted_iota(jnp.int32, (_TL, D_MODEL), 0)).astype(jnp.float32)
        col = lax.broadcasted_iota(jnp.int32, (_TL, D_MODEL), 1)
        ang = pos * jnp.exp((col & -2).astype(jnp.float32) * _SF)
        pe_s[...] = jnp.where((col & 1) == 0, jnp.sin(ang), jnp.cos(ang))

    mask = (sym_ref[0, 0, :] != PAD_IDX).astype(jnp.float32)
    out_ref[0] = emb_ref[0] + pe_s[...] * mask[:, None]


def kernel(embedded, symbol):
    B, L = symbol.shape
    sym3 = symbol.reshape(B, 1, L)
    n_l = L // _TL
    return pl.pallas_call(
        _body,
        grid=(n_l, B),
        in_specs=[
            pl.BlockSpec((1, 1, _TL), lambda i, b: (b, 0, i)),
            pl.BlockSpec((1, _TL, D_MODEL), lambda i, b: (b, i, 0)),
        ],
        out_specs=pl.BlockSpec((1, _TL, D_MODEL), lambda i, b: (b, i, 0)),
        out_shape=jax.ShapeDtypeStruct((B, L, D_MODEL), jnp.float32),
        scratch_shapes=[pltpu.VMEM((_TL, D_MODEL), jnp.float32)],
    )(sym3, embedded)


# speedup vs baseline: 4.1046x; 1.0448x over previous
import math

import jax
import jax.numpy as jnp
from jax import lax
from jax.experimental import pallas as pl
from jax.experimental.pallas import tpu as pltpu

D_MODEL = 1024
PAD_IDX = 0
_SF = -math.log(10000.0) / D_MODEL

_TL = 512


def _body(sym_ref, emb_ref, out_ref, pe_s):
    i = pl.program_id(0)
    b = pl.program_id(1)

    @pl.when(b == 0)
    def _compute_pe():
        pos = (i * _TL + lax.broadcasted_iota(jnp.int32, (_TL, D_MODEL), 0)).astype(jnp.float32)
        col = lax.broadcasted_iota(jnp.int32, (_TL, D_MODEL), 1)
        ang = pos * jnp.exp((col & -2).astype(jnp.float32) * _SF)
        ang = ang + (col & 1).astype(jnp.float32) * (math.pi / 2)
        pe_s[...] = jnp.sin(ang)

    mask = (sym_ref[0, 0, :] != PAD_IDX).astype(jnp.float32)
    out_ref[0] = emb_ref[0] + pe_s[...] * mask[:, None]


def kernel(embedded, symbol):
    B, L = symbol.shape
    sym3 = symbol.reshape(B, 1, L)
    n_l = L // _TL
    return pl.pallas_call(
        _body,
        grid=(n_l, B),
        in_specs=[
            pl.BlockSpec((1, 1, _TL), lambda i, b: (b, 0, i)),
            pl.BlockSpec((1, _TL, D_MODEL), lambda i, b: (b, i, 0)),
        ],
        out_specs=pl.BlockSpec((1, _TL, D_MODEL), lambda i, b: (b, i, 0)),
        out_shape=jax.ShapeDtypeStruct((B, L, D_MODEL), jnp.float32),
        scratch_shapes=[pltpu.VMEM((_TL, D_MODEL), jnp.float32)],
    )(sym3, embedded)
